# Initial kernel scaffold; baseline (speedup 1.0000x reference)
import jax
import jax.numpy as jnp
from jax.experimental import pallas as pl


def kernel(q_pts, s_pts, s_feats, neighbor_indices, normals, W1, b1, W2, b2, W3, b3, Wg, bg, Wv):
    raise NotImplementedError("write your pallas kernel here")


# speedup vs baseline: 1.2571x; 1.2571x over previous
import functools

import numpy as np
import jax
import jax.numpy as jnp
from jax import lax
from jax.experimental import pallas as pl
from jax.experimental.pallas import tpu as pltpu
from jax.experimental.pallas import tpu_sc as plsc

N = 10000
M = 50000
K = 16
IN_DIM = 128
OUT_DIM = 128
D = IN_DIM * 3
GW = 16
GWT = 128

NW = 32
NQW = 320
NP = NW * NQW
CQ = 4
FCH = NQW // CQ
GROWS = 128
GCH = NQW * K // GROWS

B = 128
BK = B * K
NB = NP // B

_INV_PI = float(1.0 / np.pi)
_EPS = 1e-08

@functools.cache
def _make_sc_gather():
    mesh = plsc.VectorSubcoreMesh(
        core_axis_name="c", subcore_axis_name="s", num_cores=2, num_subcores=16)
    return functools.partial(
        pl.kernel,
        out_type=(
            jax.ShapeDtypeStruct((NP, D), jnp.float32),
            jax.ShapeDtypeStruct((NP * K, GW), jnp.float32),
        ),
        mesh=mesh,
        scratch_types=[
            pltpu.VMEM((CQ * K,), jnp.int32),
            pltpu.VMEM((CQ * K, D), jnp.float32),
            pltpu.VMEM((CQ, D), jnp.float32),
            pltpu.VMEM((GROWS,), jnp.int32),
            pltpu.VMEM((GROWS, GWT), jnp.float32),
            pltpu.VMEM((GROWS, GW), jnp.float32),
            pltpu.SemaphoreType.DMA,
            pltpu.SemaphoreType.DMA,
        ],
    )(_sc_gather_body)


def _sc_gather_body(idx_hbm, feats_hbm, geom_hbm, agg_hbm, geom_out_hbm,
                    fidx, rows, acc, gidx, grows, gout, fsem, gsem):
    wid = lax.axis_index("s") * 2 + lax.axis_index("c")
    qbase = wid * NQW

    def feats_body(ch, carry):
        off = (qbase + ch * CQ) * K
        pltpu.sync_copy(idx_hbm.at[pl.ds(off, CQ * K)], fidx)
        pltpu.async_copy(feats_hbm.at[fidx], rows, fsem).wait()

        def col_body(cc, c2):
            cb = cc * 16
            for q in range(CQ):
                s = rows[q * K, pl.ds(cb, 16)]
                for r in range(1, K):
                    s = s + rows[q * K + r, pl.ds(cb, 16)]
                acc[q, pl.ds(cb, 16)] = s * np.float32(1.0 / K)
            return c2

        lax.fori_loop(0, D // 16, col_body, 0)
        pltpu.sync_copy(acc, agg_hbm.at[pl.ds(qbase + ch * CQ, CQ)])
        return carry

    lax.fori_loop(0, FCH, feats_body, 0)

    def geom_body(g, carry):
        off = qbase * K + g * GROWS
        pltpu.sync_copy(idx_hbm.at[pl.ds(off, GROWS)], gidx)
        pltpu.async_copy(geom_hbm.at[gidx], grows, gsem).wait()

        def pack_body(r, c2):
            gout[r] = grows[r, pl.ds(0, GW)]
            return c2

        lax.fori_loop(0, GROWS, pack_body, 0)
        pltpu.sync_copy(gout, geom_out_hbm.at[pl.ds(off, GROWS)])
        return carry

    lax.fori_loop(0, GCH, geom_body, 0)


def _safe_atan2(y, x):
    x_safe = jnp.where(jnp.abs(x) < _EPS, jnp.sign(x) * _EPS, x)
    return jnp.arctan2(y, x_safe)


def _tc_body(geomT_ref, qpT_ref, agg_ref, W1_ref, b1_ref, W2_ref, b2_ref,
             W3_ref, b3_ref, Wg_ref, bg_ref, Wvc_ref, RT_ref, S0T_ref,
             SKT_ref, E_ref, out_ref):
    f32 = jnp.float32
    geomT = geomT_ref[...]
    nbp = geomT[0:3, :]
    nbn = geomT[8:11, :]
    RT = RT_ref[...]
    qe = jnp.dot(qpT_ref[...], RT, preferred_element_type=f32)
    qn_small = jnp.dot(nbn, S0T_ref[...], preferred_element_type=f32)
    qn = jnp.dot(qn_small, RT, preferred_element_type=f32)

    vd = nbp - qe

    def dot3(a, b):
        return a[0:1, :] * b[0:1, :] + a[1:2, :] * b[1:2, :] + a[2:3, :] * b[2:3, :]

    def crossnorm(a, b):
        cx = a[1:2, :] * b[2:3, :] - a[2:3, :] * b[1:2, :]
        cy = a[2:3, :] * b[0:1, :] - a[0:1, :] * b[2:3, :]
        cz = a[0:1, :] * b[1:2, :] - a[1:2, :] * b[0:1, :]
        return jnp.sqrt(cx * cx + cy * cy + cz * cz)

    d = jnp.sqrt(dot3(vd, vd))
    a1 = _safe_atan2(crossnorm(qn, vd), dot3(qn, vd)) * _INV_PI
    a2 = _safe_atan2(crossnorm(nbn, vd), dot3(nbn, vd)) * _INV_PI
    a3 = _safe_atan2(crossnorm(qn, nbn), dot3(qn, nbn)) * _INV_PI
    ppfT = jnp.concatenate([d, a1, a2, a3], axis=0)

    h = jnp.dot(W1_ref[...], ppfT, preferred_element_type=f32) + b1_ref[...]
    h = jnp.maximum(h, 0.0)
    h = jnp.dot(W2_ref[...], h, preferred_element_type=f32) + b2_ref[...]
    h = jnp.maximum(h, 0.0)
    hm = jnp.dot(h, SKT_ref[...], preferred_element_type=f32)
    pm = jnp.dot(W3_ref[...], hm, preferred_element_type=f32) + b3_ref[...]
    gT = jax.nn.sigmoid(
        jnp.dot(Wg_ref[...], pm, preferred_element_type=f32) + bg_ref[...])
    ge = lax.dot_general(gT, E_ref[...], (((0,), (0,)), ((), ())),
                         preferred_element_type=f32)
    out_ref[...] = jnp.dot(agg_ref[...], Wvc_ref[...],
                           preferred_element_type=f32) * ge


_RT = np.kron(np.eye(B, dtype=np.float32), np.ones((1, K), np.float32))
_S0T = np.kron(np.eye(B, dtype=np.float32),
               np.eye(K, 1, dtype=np.float32))
_SKT = np.kron(np.eye(B, dtype=np.float32),
               np.full((K, 1), 1.0 / K, np.float32))
_E = np.kron(np.eye(OUT_DIM, dtype=np.float32), np.ones((1, 3), np.float32))
_EYE3 = np.eye(3, dtype=np.float32)


def kernel(q_pts, s_pts, s_feats, neighbor_indices, normals,
           W1, b1, W2, b2, W3, b3, Wg, bg, Wv):
    hid = W1.shape[0]
    od3 = W3.shape[0]

    idx = neighbor_indices.astype(jnp.int32).reshape(-1)
    idx_pad = jnp.concatenate(
        [idx, jnp.zeros((NP * K - N * K,), jnp.int32)])
    feats2d = s_feats.reshape(M, D)
    geom_tab = jnp.concatenate(
        [s_pts, jnp.zeros((M, 5), jnp.float32), normals,
         jnp.zeros((M, GWT - 11), jnp.float32)], axis=1)
    qpT = jnp.pad(q_pts, ((0, NP - N), (0, 0))).T

    agg, geom_rows = _make_sc_gather()(idx_pad, feats2d, geom_tab)
    geomT = geom_rows.T

    Wvc = jnp.kron(Wv.T, _EYE3)
    out2d = pl.pallas_call(
        _tc_body,
        grid=(NB,),
        in_specs=[
            pl.BlockSpec((GW, BK), lambda i: (0, i)),
            pl.BlockSpec((3, B), lambda i: (0, i)),
            pl.BlockSpec((B, D), lambda i: (i, 0)),
            pl.BlockSpec((hid, 4), lambda i: (0, 0)),
            pl.BlockSpec((hid, 1), lambda i: (0, 0)),
            pl.BlockSpec((hid, hid), lambda i: (0, 0)),
            pl.BlockSpec((hid, 1), lambda i: (0, 0)),
            pl.BlockSpec((od3, hid), lambda i: (0, 0)),
            pl.BlockSpec((od3, 1), lambda i: (0, 0)),
            pl.BlockSpec((OUT_DIM, od3), lambda i: (0, 0)),
            pl.BlockSpec((OUT_DIM, 1), lambda i: (0, 0)),
            pl.BlockSpec((D, D), lambda i: (0, 0)),
            pl.BlockSpec((B, BK), lambda i: (0, 0)),
            pl.BlockSpec((BK, B), lambda i: (0, 0)),
            pl.BlockSpec((BK, B), lambda i: (0, 0)),
            pl.BlockSpec((OUT_DIM, D), lambda i: (0, 0)),
        ],
        out_specs=pl.BlockSpec((B, D), lambda i: (i, 0)),
        out_shape=jax.ShapeDtypeStruct((NP, D), jnp.float32),
    )(geomT, qpT, agg,
      W1, b1.reshape(hid, 1), W2, b2.reshape(hid, 1),
      W3, b3.reshape(od3, 1), Wg, bg.reshape(OUT_DIM, 1),
      Wvc, jnp.asarray(_RT), jnp.asarray(_S0T), jnp.asarray(_SKT),
      jnp.asarray(_E))

    return out2d[:N].reshape(N, OUT_DIM, 3)


# speedup vs baseline: 1.8561x; 1.4765x over previous
import functools

import numpy as np
import jax
import jax.numpy as jnp
from jax import lax
from jax.experimental import pallas as pl
from jax.experimental.pallas import tpu as pltpu
from jax.experimental.pallas import tpu_sc as plsc

N = 10000
M = 50000
K = 16
IN_DIM = 128
OUT_DIM = 128
D = IN_DIM * 3
GW = 16
GWT = 128

NW = 32
NQW = 320
NP = NW * NQW
CQ = 4
EC = CQ * K
FCH = NQW // CQ

B = 128
BK = B * K
NB = NP // B

_INV_PI = float(1.0 / np.pi)
_EPS = 1e-08

@functools.cache
def _make_sc_gather():
    mesh = plsc.VectorSubcoreMesh(
        core_axis_name="c", subcore_axis_name="s", num_cores=2, num_subcores=16)
    return functools.partial(
        pl.kernel,
        out_type=(
            jax.ShapeDtypeStruct((NP * D,), jnp.float32),
            jax.ShapeDtypeStruct((NP * K * GW,), jnp.float32),
        ),
        mesh=mesh,
        scratch_types=[
            pltpu.VMEM((NQW * K,), jnp.int32),
            pltpu.VMEM((EC, D), jnp.float32),
            pltpu.VMEM((EC, D), jnp.float32),
            pltpu.VMEM((CQ * D,), jnp.float32),
            pltpu.VMEM((CQ * D,), jnp.float32),
            pltpu.VMEM((EC, GWT), jnp.float32),
            pltpu.VMEM((EC, GWT), jnp.float32),
            pltpu.VMEM((EC * GW,), jnp.float32),
            pltpu.VMEM((EC * GW,), jnp.float32),
        ] + [pltpu.SemaphoreType.DMA] * 8,
    )(_sc_gather_body)


def _sc_gather_body(idx_hbm, feats_hbm, geom_hbm, agg_hbm, geom_out_hbm,
                    idx_all, rows0, rows1, acc0, acc1, grows0, grows1,
                    gout0, gout1, fsem0, fsem1, gsem0, gsem1,
                    wsem0, wsem1, vsem0, vsem1):
    rows = (rows0, rows1)
    acc = (acc0, acc1)
    grows = (grows0, grows1)
    gout = (gout0, gout1)
    fsem = (fsem0, fsem1)
    gsem = (gsem0, gsem1)
    wsem = (wsem0, wsem1)
    vsem = (vsem0, vsem1)

    wid = lax.axis_index("s") * 2 + lax.axis_index("c")
    qbase = wid * NQW
    ebase = qbase * K

    pltpu.sync_copy(idx_hbm.at[pl.ds(ebase, NQW * K)], idx_all)

    def fgather(ch, b):
        return pltpu.make_async_copy(
            feats_hbm.at[idx_all.at[pl.ds(ch * EC, EC)]], rows[b], fsem[b])

    def ggather(ch, b):
        return pltpu.make_async_copy(
            geom_hbm.at[idx_all.at[pl.ds(ch * EC, EC)]], grows[b], gsem[b])

    def acc_wb(ch, b):
        return pltpu.make_async_copy(
            acc[b], agg_hbm.at[pl.ds((qbase + ch * CQ) * D, CQ * D)], wsem[b])

    def gout_wb(ch, b):
        return pltpu.make_async_copy(
            gout[b], geom_out_hbm.at[pl.ds((ebase + ch * EC) * GW, EC * GW)],
            vsem[b])

    for b in range(2):
        fgather(b, b).start()
        ggather(b, b).start()

    def body(g, carry):
        for b in range(2):
            ch = 2 * g + b

            fgather(ch, b).wait()

            @pl.when(g >= 1)
            def _():
                acc_wb(ch, b).wait()

            def col_body(cc, c2):
                cb = cc * 16
                for q in range(CQ):
                    s = rows[b][q * K, pl.ds(cb, 16)]
                    for r in range(1, K):
                        s = s + rows[b][q * K + r, pl.ds(cb, 16)]
                    acc[b][pl.ds(q * D + cb, 16)] = s * np.float32(1.0 / K)
                return c2

            lax.fori_loop(0, D // 16, col_body, 0)
            acc_wb(ch, b).start()

            ggather(ch, b).wait()

            @pl.when(g >= 1)
            def _():
                gout_wb(ch, b).wait()

            def pack_body(r, c2):
                gout[b][pl.ds(r * GW, GW)] = grows[b][r, pl.ds(0, GW)]
                return c2

            lax.fori_loop(0, EC, pack_body, 0)
            gout_wb(ch, b).start()

            @pl.when(ch + 2 < FCH)
            def _():
                fgather(ch + 2, b).start()
                ggather(ch + 2, b).start()
        return carry

    lax.fori_loop(0, FCH // 2, body, 0)

    for b in range(2):
        acc_wb(FCH - 2 + b, b).wait()
        gout_wb(FCH - 2 + b, b).wait()


def _safe_atan2(y, x):
    x_safe = jnp.where(jnp.abs(x) < _EPS, jnp.sign(x) * _EPS, x)
    return jnp.arctan2(y, x_safe)


def _tc_body(geomT_ref, qpT_ref, agg_ref, W1_ref, b1_ref, W2_ref, b2_ref,
             W3_ref, b3_ref, Wg_ref, bg_ref, Wvc_ref, RT_ref, S0T_ref,
             SKT_ref, E_ref, out_ref):
    f32 = jnp.float32
    geomT = geomT_ref[...]
    nbp = geomT[0:3, :]
    nbn = geomT[8:11, :]
    RT = RT_ref[...]
    qe = jnp.dot(qpT_ref[...], RT, preferred_element_type=f32)
    qn_small = jnp.dot(nbn, S0T_ref[...], preferred_element_type=f32)
    qn = jnp.dot(qn_small, RT, preferred_element_type=f32)

    vd = nbp - qe

    def dot3(a, b):
        return a[0:1, :] * b[0:1, :] + a[1:2, :] * b[1:2, :] + a[2:3, :] * b[2:3, :]

    def crossnorm(a, b):
        cx = a[1:2, :] * b[2:3, :] - a[2:3, :] * b[1:2, :]
        cy = a[2:3, :] * b[0:1, :] - a[0:1, :] * b[2:3, :]
        cz = a[0:1, :] * b[1:2, :] - a[1:2, :] * b[0:1, :]
        return jnp.sqrt(cx * cx + cy * cy + cz * cz)

    d = jnp.sqrt(dot3(vd, vd))
    a1 = _safe_atan2(crossnorm(qn, vd), dot3(qn, vd)) * _INV_PI
    a2 = _safe_atan2(crossnorm(nbn, vd), dot3(nbn, vd)) * _INV_PI
    a3 = _safe_atan2(crossnorm(qn, nbn), dot3(qn, nbn)) * _INV_PI
    ppfT = jnp.concatenate([d, a1, a2, a3], axis=0)

    h = jnp.dot(W1_ref[...], ppfT, preferred_element_type=f32) + b1_ref[...]
    h = jnp.maximum(h, 0.0)
    h = jnp.dot(W2_ref[...], h, preferred_element_type=f32) + b2_ref[...]
    h = jnp.maximum(h, 0.0)
    hm = jnp.dot(h, SKT_ref[...], preferred_element_type=f32)
    pm = jnp.dot(W3_ref[...], hm, preferred_element_type=f32) + b3_ref[...]
    gT = jax.nn.sigmoid(
        jnp.dot(Wg_ref[...], pm, preferred_element_type=f32) + bg_ref[...])
    ge = lax.dot_general(gT, E_ref[...], (((0,), (0,)), ((), ())),
                         preferred_element_type=f32)
    out_ref[...] = jnp.dot(agg_ref[...], Wvc_ref[...],
                           preferred_element_type=f32) * ge


_RT = np.kron(np.eye(B, dtype=np.float32), np.ones((1, K), np.float32))
_S0T = np.kron(np.eye(B, dtype=np.float32),
               np.eye(K, 1, dtype=np.float32))
_SKT = np.kron(np.eye(B, dtype=np.float32),
               np.full((K, 1), 1.0 / K, np.float32))
_E = np.kron(np.eye(OUT_DIM, dtype=np.float32), np.ones((1, 3), np.float32))
_EYE3 = np.eye(3, dtype=np.float32)


def kernel(q_pts, s_pts, s_feats, neighbor_indices, normals,
           W1, b1, W2, b2, W3, b3, Wg, bg, Wv):
    hid = W1.shape[0]
    od3 = W3.shape[0]

    idx = neighbor_indices.astype(jnp.int32).reshape(-1)
    idx_pad = jnp.concatenate(
        [idx, jnp.zeros((NP * K - N * K,), jnp.int32)])
    feats2d = s_feats.reshape(M, D)
    geom_tab = jnp.concatenate(
        [s_pts, jnp.zeros((M, 5), jnp.float32), normals,
         jnp.zeros((M, GWT - 11), jnp.float32)], axis=1)
    qpT = jnp.pad(q_pts, ((0, NP - N), (0, 0))).T

    agg_flat, geom_flat = _make_sc_gather()(idx_pad, feats2d, geom_tab)
    agg = agg_flat.reshape(NP, D)
    geomT = geom_flat.reshape(NP * K, GW).T

    Wvc = jnp.kron(Wv.T, _EYE3)
    out2d = pl.pallas_call(
        _tc_body,
        grid=(NB,),
        in_specs=[
            pl.BlockSpec((GW, BK), lambda i: (0, i)),
            pl.BlockSpec((3, B), lambda i: (0, i)),
            pl.BlockSpec((B, D), lambda i: (i, 0)),
            pl.BlockSpec((hid, 4), lambda i: (0, 0)),
            pl.BlockSpec((hid, 1), lambda i: (0, 0)),
            pl.BlockSpec((hid, hid), lambda i: (0, 0)),
            pl.BlockSpec((hid, 1), lambda i: (0, 0)),
            pl.BlockSpec((od3, hid), lambda i: (0, 0)),
            pl.BlockSpec((od3, 1), lambda i: (0, 0)),
            pl.BlockSpec((OUT_DIM, od3), lambda i: (0, 0)),
            pl.BlockSpec((OUT_DIM, 1), lambda i: (0, 0)),
            pl.BlockSpec((D, D), lambda i: (0, 0)),
            pl.BlockSpec((B, BK), lambda i: (0, 0)),
            pl.BlockSpec((BK, B), lambda i: (0, 0)),
            pl.BlockSpec((BK, B), lambda i: (0, 0)),
            pl.BlockSpec((OUT_DIM, D), lambda i: (0, 0)),
        ],
        out_specs=pl.BlockSpec((B, D), lambda i: (i, 0)),
        out_shape=jax.ShapeDtypeStruct((NP, D), jnp.float32),
    )(geomT, qpT, agg,
      W1, b1.reshape(hid, 1), W2, b2.reshape(hid, 1),
      W3, b3.reshape(od3, 1), Wg, bg.reshape(OUT_DIM, 1),
      Wvc, jnp.asarray(_RT), jnp.asarray(_S0T), jnp.asarray(_SKT),
      jnp.asarray(_E))

    return out2d[:N].reshape(N, OUT_DIM, 3)


# speedup vs baseline: 2.0977x; 1.1301x over previous
import functools

import numpy as np
import jax
import jax.numpy as jnp
from jax import lax
from jax.experimental import pallas as pl
from jax.experimental.pallas import tpu as pltpu
from jax.experimental.pallas import tpu_sc as plsc

N = 10000
M = 50000
K = 16
IN_DIM = 128
OUT_DIM = 128
D = IN_DIM * 3
GW = 16
GWT = 128

NW = 32
NQW = 320
NP = NW * NQW
CQ = 4
EC = CQ * K
FCH = NQW // CQ

B = 128
BK = B * K
NB = NP // B

_INV_PI = float(1.0 / np.pi)
_EPS = 1e-08

@functools.cache
def _make_sc_gather():
    mesh = plsc.VectorSubcoreMesh(
        core_axis_name="c", subcore_axis_name="s", num_cores=2, num_subcores=16)
    return functools.partial(
        pl.kernel,
        out_type=(
            jax.ShapeDtypeStruct((NP * D,), jnp.float32),
            jax.ShapeDtypeStruct((NP * K * GW,), jnp.float32),
        ),
        mesh=mesh,
        scratch_types=[
            pltpu.VMEM((NQW * K,), jnp.int32),
            pltpu.VMEM((EC, D), jnp.float32),
            pltpu.VMEM((EC, D), jnp.float32),
            pltpu.VMEM((CQ * D,), jnp.float32),
            pltpu.VMEM((CQ * D,), jnp.float32),
            pltpu.VMEM((EC, GWT), jnp.float32),
            pltpu.VMEM((EC, GWT), jnp.float32),
            pltpu.VMEM((EC * GW,), jnp.float32),
            pltpu.VMEM((EC * GW,), jnp.float32),
        ] + [pltpu.SemaphoreType.DMA] * 8,
    )(_sc_gather_body)


def _sc_gather_body(idx_hbm, feats_hbm, geom_hbm, agg_hbm, geom_out_hbm,
                    idx_all, rows0, rows1, acc0, acc1, grows0, grows1,
                    gout0, gout1, fsem0, fsem1, gsem0, gsem1,
                    wsem0, wsem1, vsem0, vsem1):
    rows = (rows0, rows1)
    acc = (acc0, acc1)
    grows = (grows0, grows1)
    gout = (gout0, gout1)
    fsem = (fsem0, fsem1)
    gsem = (gsem0, gsem1)
    wsem = (wsem0, wsem1)
    vsem = (vsem0, vsem1)

    wid = lax.axis_index("s") * 2 + lax.axis_index("c")
    qbase = wid * NQW
    ebase = qbase * K

    pltpu.sync_copy(idx_hbm.at[pl.ds(ebase, NQW * K)], idx_all)

    def fgather(ch, b):
        return pltpu.make_async_copy(
            feats_hbm.at[idx_all.at[pl.ds(ch * EC, EC)]], rows[b], fsem[b])

    def ggather(ch, b):
        return pltpu.make_async_copy(
            geom_hbm.at[idx_all.at[pl.ds(ch * EC, EC)]], grows[b], gsem[b])

    def acc_wb(ch, b):
        return pltpu.make_async_copy(
            acc[b], agg_hbm.at[pl.ds((qbase + ch * CQ) * D, CQ * D)], wsem[b])

    def gout_wb(ch, b):
        return pltpu.make_async_copy(
            gout[b], geom_out_hbm.at[pl.ds((ebase + ch * EC) * GW, EC * GW)],
            vsem[b])

    for b in range(2):
        fgather(b, b).start()
        ggather(b, b).start()

    def body(g, carry):
        for b in range(2):
            ch = 2 * g + b

            fgather(ch, b).wait()

            @pl.when(g >= 1)
            def _():
                acc_wb(ch, b).wait()

            def col_body(cc, c2):
                cb = cc * 16
                for q in range(CQ):
                    s = rows[b][q * K, pl.ds(cb, 16)]
                    for r in range(1, K):
                        s = s + rows[b][q * K + r, pl.ds(cb, 16)]
                    acc[b][pl.ds(q * D + cb, 16)] = s * np.float32(1.0 / K)
                return c2

            lax.fori_loop(0, D // 16, col_body, 0)
            acc_wb(ch, b).start()

            ggather(ch, b).wait()

            @pl.when(g >= 1)
            def _():
                gout_wb(ch, b).wait()

            def pack_body(r, c2):
                gout[b][pl.ds(r * GW, GW)] = grows[b][r, pl.ds(0, GW)]
                return c2

            lax.fori_loop(0, EC, pack_body, 0)
            gout_wb(ch, b).start()

            @pl.when(ch + 2 < FCH)
            def _():
                fgather(ch + 2, b).start()
                ggather(ch + 2, b).start()
        return carry

    lax.fori_loop(0, FCH // 2, body, 0)

    for b in range(2):
        acc_wb(FCH - 2 + b, b).wait()
        gout_wb(FCH - 2 + b, b).wait()


def _safe_atan2(y, x):
    x_safe = jnp.where(jnp.abs(x) < _EPS, jnp.sign(x) * _EPS, x)
    return jnp.arctan2(y, x_safe)


def _tc_body(geomT_ref, qpT_ref, agg_ref, W1_ref, b1_ref, W2_ref, b2_ref,
             W3_ref, b3_ref, Wg_ref, bg_ref, Wvc_ref, RT_ref, S0T_ref,
             SKT_ref, E_ref, out_ref):
    f32 = jnp.float32
    geomT = geomT_ref[...]
    nbp = geomT[0:3, :]
    nbn = geomT[8:11, :]
    RT = RT_ref[...]
    qe = jnp.dot(qpT_ref[...], RT, preferred_element_type=f32)
    qn_small = jnp.dot(nbn, S0T_ref[...], preferred_element_type=f32)
    qn = jnp.dot(qn_small, RT, preferred_element_type=f32)

    vd = nbp - qe

    def dot3(a, b):
        return a[0:1, :] * b[0:1, :] + a[1:2, :] * b[1:2, :] + a[2:3, :] * b[2:3, :]

    def crossnorm(a, b):
        cx = a[1:2, :] * b[2:3, :] - a[2:3, :] * b[1:2, :]
        cy = a[2:3, :] * b[0:1, :] - a[0:1, :] * b[2:3, :]
        cz = a[0:1, :] * b[1:2, :] - a[1:2, :] * b[0:1, :]
        return jnp.sqrt(cx * cx + cy * cy + cz * cz)

    d = jnp.sqrt(dot3(vd, vd))
    a1 = _safe_atan2(crossnorm(qn, vd), dot3(qn, vd)) * _INV_PI
    a2 = _safe_atan2(crossnorm(nbn, vd), dot3(nbn, vd)) * _INV_PI
    a3 = _safe_atan2(crossnorm(qn, nbn), dot3(qn, nbn)) * _INV_PI
    ppfT = jnp.concatenate([d, a1, a2, a3], axis=0)

    h = jnp.dot(W1_ref[...], ppfT, preferred_element_type=f32) + b1_ref[...]
    h = jnp.maximum(h, 0.0)
    h = jnp.dot(W2_ref[...], h, preferred_element_type=f32) + b2_ref[...]
    h = jnp.maximum(h, 0.0)
    hm = jnp.dot(h, SKT_ref[...], preferred_element_type=f32)
    pm = jnp.dot(W3_ref[...], hm, preferred_element_type=f32) + b3_ref[...]
    gT = jax.nn.sigmoid(
        jnp.dot(Wg_ref[...], pm, preferred_element_type=f32) + bg_ref[...])
    ge = lax.dot_general(gT, E_ref[...], (((0,), (0,)), ((), ())),
                         preferred_element_type=f32)
    out_ref[...] = jnp.dot(agg_ref[...], Wvc_ref[...],
                           preferred_element_type=f32) * ge


_RT = np.kron(np.eye(B, dtype=np.float32), np.ones((1, K), np.float32))
_S0T = np.kron(np.eye(B, dtype=np.float32),
               np.eye(K, 1, dtype=np.float32))
_SKT = np.kron(np.eye(B, dtype=np.float32),
               np.full((K, 1), 1.0 / K, np.float32))
_E = np.kron(np.eye(OUT_DIM, dtype=np.float32), np.ones((1, 3), np.float32))
_EYE3 = np.eye(3, dtype=np.float32)


def kernel(q_pts, s_pts, s_feats, neighbor_indices, normals,
           W1, b1, W2, b2, W3, b3, Wg, bg, Wv):
    hid = W1.shape[0]
    od3 = W3.shape[0]

    idx = neighbor_indices.astype(jnp.int32).reshape(-1)
    idx_pad = jnp.concatenate(
        [idx, jnp.zeros((NP * K - N * K,), jnp.int32)])
    feats2d = s_feats.transpose(0, 2, 1).reshape(M, D)
    geom_tab = jnp.concatenate(
        [s_pts, jnp.zeros((M, 5), jnp.float32), normals,
         jnp.zeros((M, GWT - 11), jnp.float32)], axis=1)
    qpT = jnp.pad(q_pts, ((0, NP - N), (0, 0))).T

    agg_flat, geom_flat = _make_sc_gather()(idx_pad, feats2d, geom_tab)
    agg = agg_flat.reshape(NP, D)
    geomT = geom_flat.reshape(NP * K, GW).T

    Wvc = (jnp.asarray(_EYE3)[:, None, None, :]
           * Wv.T[None, :, :, None]).reshape(D, D)
    out2d = pl.pallas_call(
        _tc_body,
        grid=(NB,),
        in_specs=[
            pl.BlockSpec((GW, BK), lambda i: (0, i)),
            pl.BlockSpec((3, B), lambda i: (0, i)),
            pl.BlockSpec((B, D), lambda i: (i, 0)),
            pl.BlockSpec((hid, 4), lambda i: (0, 0)),
            pl.BlockSpec((hid, 1), lambda i: (0, 0)),
            pl.BlockSpec((hid, hid), lambda i: (0, 0)),
            pl.BlockSpec((hid, 1), lambda i: (0, 0)),
            pl.BlockSpec((od3, hid), lambda i: (0, 0)),
            pl.BlockSpec((od3, 1), lambda i: (0, 0)),
            pl.BlockSpec((OUT_DIM, od3), lambda i: (0, 0)),
            pl.BlockSpec((OUT_DIM, 1), lambda i: (0, 0)),
            pl.BlockSpec((D, D), lambda i: (0, 0)),
            pl.BlockSpec((B, BK), lambda i: (0, 0)),
            pl.BlockSpec((BK, B), lambda i: (0, 0)),
            pl.BlockSpec((BK, B), lambda i: (0, 0)),
            pl.BlockSpec((OUT_DIM, D), lambda i: (0, 0)),
        ],
        out_specs=pl.BlockSpec((B, D), lambda i: (i, 0)),
        out_shape=jax.ShapeDtypeStruct((NP, D), jnp.float32),
    )(geomT, qpT, agg,
      W1, b1.reshape(hid, 1), W2, b2.reshape(hid, 1),
      W3, b3.reshape(od3, 1), Wg, bg.reshape(OUT_DIM, 1),
      Wvc, jnp.asarray(_RT), jnp.asarray(_S0T), jnp.asarray(_SKT),
      jnp.asarray(_E))

    return out2d[:N].reshape(N, OUT_DIM, 3)
